# baseline (speedup 1.0000x reference)
import functools

import jax
import jax.numpy as jnp
from jax import lax
from jax.experimental import pallas as pl
from jax.experimental.pallas import tpu as pltpu
from jax.experimental.pallas import tpu_sc as plsc

N = 10000
NC, NS, NW = 2, 16, 32
RPS = N // NS
EPW = 10000

_MESH = plsc.VectorSubcoreMesh(
    core_axis_name="c", subcore_axis_name="s", num_cores=NC, num_subcores=NS)

_SC_PARAMS = pltpu.CompilerParams(
    use_tc_tiling_on_sc=False, needs_layout_passes=False,
    disable_bounds_checks=True)


def _make_deg_kernel(ch):
  nch = EPW // ch

  @functools.partial(
      pl.kernel, mesh=_MESH,
      compiler_params=_SC_PARAMS,
      out_type=jax.ShapeDtypeStruct((NC, N, 16), jnp.float32),
      scratch_types=[
          pltpu.VMEM((nch, ch), jnp.int32),
          pltpu.VMEM((ch, 16), jnp.float32),
          pltpu.VMEM_SHARED((N, 16), jnp.float32),
      ],
  )
  def deg_kernel(dst_hbm, zeros_hbm, out_hbm, idx_v, ones_v, deg_sh):
    cid = lax.axis_index("c")
    sid = lax.axis_index("s")
    wid = cid * NS + sid
    pltpu.sync_copy(zeros_hbm, deg_sh.at[pl.ds(sid * RPS, RPS)])
    pltpu.sync_copy(dst_hbm.at[wid], idx_v)
    for r in range(ch):
      ones_v[r, :] = jnp.full((16,), 1.0, jnp.float32)
    plsc.subcore_barrier()

    def body(c, carry):
      pltpu.sync_copy(ones_v, deg_sh.at[idx_v.at[c]], add=True)
      return carry

    lax.fori_loop(0, nch, body, 0)
    plsc.subcore_barrier()
    pltpu.sync_copy(deg_sh.at[pl.ds(sid * RPS, RPS)],
                    out_hbm.at[cid, pl.ds(sid * RPS, RPS)])

  return deg_kernel


def _make_edge_kernel(d, ch):
  nch = EPW // ch

  @functools.partial(
      pl.kernel, mesh=_MESH,
      compiler_params=_SC_PARAMS,
      out_type=jax.ShapeDtypeStruct((NC, N, d), jnp.float32),
      scratch_types=[
          pltpu.VMEM((nch, ch), jnp.int32),
          pltpu.VMEM((nch, ch), jnp.int32),
          pltpu.VMEM((ch, d), jnp.float32),
          pltpu.VMEM((ch, d), jnp.float32),
          pltpu.VMEM((ch, d), jnp.float32),
          pltpu.VMEM((ch, d), jnp.float32),
          pltpu.VMEM_SHARED((N, d), jnp.float32),
          pltpu.SemaphoreType.DMA,
          pltpu.SemaphoreType.DMA,
      ],
  )
  def edge_kernel(y_hbm, src_hbm, dst_hbm, zeros_hbm, out_hbm,
                  src_v, dst_v, rows0, rows1, rows2, rows3, acc_sh,
                  sem_g, sem_s):
    cid = lax.axis_index("c")
    sid = lax.axis_index("s")
    wid = cid * NS + sid
    pltpu.sync_copy(zeros_hbm, acc_sh.at[pl.ds(sid * RPS, RPS)])
    pltpu.sync_copy(src_hbm.at[wid], src_v)
    pltpu.sync_copy(dst_hbm.at[wid], dst_v)
    plsc.subcore_barrier()

    bufs = (rows0, rows1, rows2, rows3)
    pltpu.async_copy(y_hbm.at[src_v.at[0]], bufs[0], sem_g)
    pltpu.async_copy(y_hbm.at[src_v.at[1]], bufs[1], sem_g)

    def step(c, b):
      @pl.when(c >= 2)
      def _():
        pltpu.make_async_copy(bufs[(b + 2) % 4],
                              acc_sh.at[dst_v.at[c - 2]], sem_s).wait()

      @pl.when(c + 2 < nch)
      def _():
        pltpu.async_copy(y_hbm.at[src_v.at[c + 2]], bufs[(b + 2) % 4], sem_g)

      pltpu.make_async_copy(y_hbm.at[src_v.at[c]], bufs[b], sem_g).wait()
      pltpu.async_copy(bufs[b], acc_sh.at[dst_v.at[c]], sem_s, add=True)

    def body(k, carry):
      c = k * 4
      for off in range(4):
        step(c + off, off)
      return carry

    lax.fori_loop(0, nch // 4, body, 0)
    for c in range((nch // 4) * 4, nch):
      step(c, c % 4)
    pltpu.make_async_copy(bufs[(nch - 2) % 4],
                          acc_sh.at[dst_v.at[nch - 2]], sem_s).wait()
    pltpu.make_async_copy(bufs[(nch - 1) % 4],
                          acc_sh.at[dst_v.at[nch - 1]], sem_s).wait()
    plsc.subcore_barrier()
    pltpu.sync_copy(acc_sh.at[pl.ds(sid * RPS, RPS)],
                    out_hbm.at[cid, pl.ds(sid * RPS, RPS)])

  return edge_kernel


def _make_decode_kernel(d, ch):
  nch = EPW // ch

  nbuf = 3

  @functools.partial(
      pl.kernel, mesh=_MESH,
      compiler_params=_SC_PARAMS,
      out_type=jax.ShapeDtypeStruct((NW, nch, ch), jnp.float32),
      scratch_types=[
          pltpu.VMEM((nch, ch), jnp.int32),
          pltpu.VMEM((nch, ch), jnp.int32),
          [pltpu.VMEM((ch, d), jnp.float32) for _ in range(nbuf)],
          [pltpu.VMEM((ch, d), jnp.float32) for _ in range(nbuf)],
          pltpu.VMEM((nch, ch), jnp.float32),
          pltpu.SemaphoreType.DMA,
      ],
  )
  def decode_kernel(z_hbm, src_hbm, dst_hbm, out_hbm,
                    src_v, dst_v, zrs, zcs, out_v, sem_g):
    cid = lax.axis_index("c")
    sid = lax.axis_index("s")
    wid = cid * NS + sid
    pltpu.sync_copy(src_hbm.at[wid], src_v)
    pltpu.sync_copy(dst_hbm.at[wid], dst_v)

    for p in range(nbuf - 1):
      pltpu.async_copy(z_hbm.at[src_v.at[p]], zrs[p], sem_g)
      pltpu.async_copy(z_hbm.at[dst_v.at[p]], zcs[p], sem_g)

    lanes = lax.iota(jnp.int32, 16)
    colh = [((lanes + u) & 3) * 16 for u in range(4)]

    def step(c, b):
      zr_cur, zc_cur = zrs[b], zcs[b]
      nb = (b + nbuf - 1) % nbuf

      @pl.when(c + nbuf - 1 < nch)
      def _():
        pltpu.async_copy(z_hbm.at[src_v.at[c + nbuf - 1]], zrs[nb], sem_g)
        pltpu.async_copy(z_hbm.at[dst_v.at[c + nbuf - 1]], zcs[nb], sem_g)

      pltpu.make_async_copy(z_hbm.at[src_v.at[c]], zr_cur, sem_g).wait()
      pltpu.make_async_copy(z_hbm.at[dst_v.at[c]], zc_cur, sem_g).wait()
      def group(g, carry):
        rows = lanes + g * 16
        accs = [jnp.zeros((16,), jnp.float32) for _ in range(8)]
        for t in range(16):
          colt = (lanes + t) & 15
          for u in range(d // 16):
            col = colt + colh[u]
            a = u + 4 * (t % 2)
            accs[a] = accs[a] + (plsc.load_gather(zr_cur, [rows, col]) *
                                 plsc.load_gather(zc_cur, [rows, col]))
        s0 = (accs[0] + accs[1]) + (accs[2] + accs[3])
        s1 = (accs[4] + accs[5]) + (accs[6] + accs[7])
        out_v[c, pl.ds(g * 16, 16)] = s0 + s1
        return carry

      lax.fori_loop(0, ch // 16, group, 0)

    def body(k, carry):
      c = k * nbuf
      for off in range(nbuf):
        step(c + off, off)
      return carry

    lax.fori_loop(0, nch // nbuf, body, 0)
    for c in range((nch // nbuf) * nbuf, nch):
      step(c, c % nbuf)
    pltpu.sync_copy(out_v, out_hbm.at[wid])

  return decode_kernel


def _tc_y1(x, W1, deg0, deg1):
  blk = 1000
  grid = N // blk

  def body(xb, wb, d0b, d1b, yb):
    dinv = lax.rsqrt(d0b[...] + d1b[...] + 1.0)
    yb[...] = dinv * jnp.dot(xb[...], wb[...],
                             preferred_element_type=jnp.float32)

  return pl.pallas_call(
      body,
      grid=(grid,),
      in_specs=[
          pl.BlockSpec((blk, 128), lambda i: (i, 0)),
          pl.BlockSpec((128, 128), lambda i: (0, 0)),
          pl.BlockSpec((blk, 1), lambda i: (i, 0)),
          pl.BlockSpec((blk, 1), lambda i: (i, 0)),
      ],
      out_specs=pl.BlockSpec((blk, 128), lambda i: (i, 0)),
      out_shape=jax.ShapeDtypeStruct((N, 128), jnp.float32),
  )(x, W1, deg0, deg1)


def _tc_layer2_in(S1p, y1, deg0, deg1, b1, W2):
  blk = 1000
  grid = N // blk

  def body(sb, yb, d0b, d1b, bb, wb, ob):
    dinv = lax.rsqrt(d0b[...] + d1b[...] + 1.0)
    h = jax.nn.relu(dinv * (sb[0] + sb[1] + yb[...]) + bb[...])
    ob[...] = dinv * jnp.dot(h, wb[...], preferred_element_type=jnp.float32)

  return pl.pallas_call(
      body,
      grid=(grid,),
      in_specs=[
          pl.BlockSpec((2, blk, 128), lambda i: (0, i, 0)),
          pl.BlockSpec((blk, 128), lambda i: (i, 0)),
          pl.BlockSpec((blk, 1), lambda i: (i, 0)),
          pl.BlockSpec((blk, 1), lambda i: (i, 0)),
          pl.BlockSpec((1, 128), lambda i: (0, 0)),
          pl.BlockSpec((128, 64), lambda i: (0, 0)),
      ],
      out_specs=pl.BlockSpec((blk, 64), lambda i: (i, 0)),
      out_shape=jax.ShapeDtypeStruct((N, 64), jnp.float32),
  )(S1p, y1, deg0, deg1, b1, W2)


def _tc_z(S2p, y2, deg0, deg1, b2):
  blk = 1000
  grid = N // blk

  def body(sb, yb, d0b, d1b, bb, zb):
    dinv = lax.rsqrt(d0b[...] + d1b[...] + 1.0)
    zb[...] = dinv * (sb[0] + sb[1] + yb[...]) + bb[...]

  return pl.pallas_call(
      body,
      grid=(grid,),
      in_specs=[
          pl.BlockSpec((2, blk, 64), lambda i: (0, i, 0)),
          pl.BlockSpec((blk, 64), lambda i: (i, 0)),
          pl.BlockSpec((blk, 1), lambda i: (i, 0)),
          pl.BlockSpec((blk, 1), lambda i: (i, 0)),
          pl.BlockSpec((1, 64), lambda i: (0, 0)),
      ],
      out_specs=pl.BlockSpec((blk, 64), lambda i: (i, 0)),
      out_shape=jax.ShapeDtypeStruct((N, 64), jnp.float32),
  )(S2p, y2, deg0, deg1, b2)


_deg_kernel = _make_deg_kernel(80)
_edge_kernel_128 = _make_edge_kernel(128, 40)
_edge_kernel_64 = _make_edge_kernel(64, 80)
_decode_kernel = _make_decode_kernel(64, 80)


def kernel(x, edge_index, W1, b1, W2, b2):
  e = edge_index.shape[1]
  src80 = edge_index[0].reshape(NW, EPW // 80, 80)
  dst80 = edge_index[1].reshape(NW, EPW // 80, 80)
  src40 = edge_index[0].reshape(NW, EPW // 40, 40)
  dst40 = edge_index[1].reshape(NW, EPW // 40, 40)

  deg_part = _deg_kernel(dst80, jnp.zeros((RPS, 16), jnp.float32))
  deg0 = deg_part[0, :, 0:1]
  deg1 = deg_part[1, :, 0:1]

  y1 = _tc_y1(x, W1, deg0, deg1)
  S1p = _edge_kernel_128(y1, src40, dst40, jnp.zeros((RPS, 128), jnp.float32))
  y2 = _tc_layer2_in(S1p, y1, deg0, deg1, b1.reshape(1, 128), W2)
  S2p = _edge_kernel_64(y2, src80, dst80, jnp.zeros((RPS, 64), jnp.float32))
  z = _tc_z(S2p, y2, deg0, deg1, b2.reshape(1, 64))

  scores = _decode_kernel(z, src80, dst80)
  return scores.reshape(e)


# speedup vs baseline: 1.0399x; 1.0399x over previous
import functools

import jax
import jax.numpy as jnp
from jax import lax
from jax.experimental import pallas as pl
from jax.experimental.pallas import tpu as pltpu
from jax.experimental.pallas import tpu_sc as plsc

N = 10000
NC, NS, NW = 2, 16, 32
RPS = N // NS
EPW = 10000

_MESH = plsc.VectorSubcoreMesh(
    core_axis_name="c", subcore_axis_name="s", num_cores=NC, num_subcores=NS)

_SC_PARAMS = pltpu.CompilerParams(
    use_tc_tiling_on_sc=False, needs_layout_passes=False,
    disable_bounds_checks=True)


def _make_deg_kernel(ch):
  nch = EPW // ch

  @functools.partial(
      pl.kernel, mesh=_MESH,
      compiler_params=_SC_PARAMS,
      out_type=jax.ShapeDtypeStruct((NC, N, 16), jnp.float32),
      scratch_types=[
          pltpu.VMEM((nch, ch), jnp.int32),
          pltpu.VMEM((ch, 16), jnp.float32),
          pltpu.VMEM_SHARED((N, 16), jnp.float32),
      ],
  )
  def deg_kernel(dst_hbm, zeros_hbm, out_hbm, idx_v, ones_v, deg_sh):
    cid = lax.axis_index("c")
    sid = lax.axis_index("s")
    wid = cid * NS + sid
    pltpu.sync_copy(zeros_hbm, deg_sh.at[pl.ds(sid * RPS, RPS)])
    pltpu.sync_copy(dst_hbm.at[wid], idx_v)
    for r in range(ch):
      ones_v[r, :] = jnp.full((16,), 1.0, jnp.float32)
    plsc.subcore_barrier()

    def body(c, carry):
      pltpu.sync_copy(ones_v, deg_sh.at[idx_v.at[c]], add=True)
      return carry

    lax.fori_loop(0, nch, body, 0)
    plsc.subcore_barrier()
    pltpu.sync_copy(deg_sh.at[pl.ds(sid * RPS, RPS)],
                    out_hbm.at[cid, pl.ds(sid * RPS, RPS)])

  return deg_kernel


def _make_edge_kernel(d, ch):
  nch = EPW // ch

  @functools.partial(
      pl.kernel, mesh=_MESH,
      compiler_params=_SC_PARAMS,
      out_type=jax.ShapeDtypeStruct((NC, N, d), jnp.float32),
      scratch_types=[
          pltpu.VMEM((nch, ch), jnp.int32),
          pltpu.VMEM((nch, ch), jnp.int32),
          pltpu.VMEM((ch, d), jnp.float32),
          pltpu.VMEM((ch, d), jnp.float32),
          pltpu.VMEM((ch, d), jnp.float32),
          pltpu.VMEM((ch, d), jnp.float32),
          pltpu.VMEM_SHARED((N, d), jnp.float32),
          pltpu.SemaphoreType.DMA,
          pltpu.SemaphoreType.DMA,
      ],
  )
  def edge_kernel(y_hbm, src_hbm, dst_hbm, zeros_hbm, out_hbm,
                  src_v, dst_v, rows0, rows1, rows2, rows3, acc_sh,
                  sem_g, sem_s):
    cid = lax.axis_index("c")
    sid = lax.axis_index("s")
    wid = cid * NS + sid
    pltpu.sync_copy(zeros_hbm, acc_sh.at[pl.ds(sid * RPS, RPS)])
    pltpu.sync_copy(src_hbm.at[wid], src_v)
    pltpu.sync_copy(dst_hbm.at[wid], dst_v)
    plsc.subcore_barrier()

    bufs = (rows0, rows1, rows2, rows3)
    pltpu.async_copy(y_hbm.at[src_v.at[0]], bufs[0], sem_g)
    pltpu.async_copy(y_hbm.at[src_v.at[1]], bufs[1], sem_g)

    def step(c, b):
      @pl.when(c >= 2)
      def _():
        pltpu.make_async_copy(bufs[(b + 2) % 4],
                              acc_sh.at[dst_v.at[c - 2]], sem_s).wait()

      @pl.when(c + 2 < nch)
      def _():
        pltpu.async_copy(y_hbm.at[src_v.at[c + 2]], bufs[(b + 2) % 4], sem_g)

      pltpu.make_async_copy(y_hbm.at[src_v.at[c]], bufs[b], sem_g).wait()
      pltpu.async_copy(bufs[b], acc_sh.at[dst_v.at[c]], sem_s, add=True)

    def body(k, carry):
      c = k * 4
      for off in range(4):
        step(c + off, off)
      return carry

    lax.fori_loop(0, nch // 4, body, 0)
    for c in range((nch // 4) * 4, nch):
      step(c, c % 4)
    pltpu.make_async_copy(bufs[(nch - 2) % 4],
                          acc_sh.at[dst_v.at[nch - 2]], sem_s).wait()
    pltpu.make_async_copy(bufs[(nch - 1) % 4],
                          acc_sh.at[dst_v.at[nch - 1]], sem_s).wait()
    plsc.subcore_barrier()
    pltpu.sync_copy(acc_sh.at[pl.ds(sid * RPS, RPS)],
                    out_hbm.at[cid, pl.ds(sid * RPS, RPS)])

  return edge_kernel


def _make_decode_kernel(d, ch):
  nch = EPW // ch

  nbuf = 4

  @functools.partial(
      pl.kernel, mesh=_MESH,
      compiler_params=_SC_PARAMS,
      out_type=jax.ShapeDtypeStruct((NW, nch, ch), jnp.float32),
      scratch_types=[
          pltpu.VMEM((nch, ch), jnp.int32),
          pltpu.VMEM((nch, ch), jnp.int32),
          [pltpu.VMEM((ch, d), jnp.float32) for _ in range(nbuf)],
          [pltpu.VMEM((ch, d), jnp.float32) for _ in range(nbuf)],
          pltpu.VMEM((nch, ch), jnp.float32),
          pltpu.SemaphoreType.DMA,
      ],
  )
  def decode_kernel(z_hbm, src_hbm, dst_hbm, out_hbm,
                    src_v, dst_v, zrs, zcs, out_v, sem_g):
    cid = lax.axis_index("c")
    sid = lax.axis_index("s")
    wid = cid * NS + sid
    pltpu.sync_copy(src_hbm.at[wid], src_v)
    pltpu.sync_copy(dst_hbm.at[wid], dst_v)

    for p in range(nbuf - 1):
      pltpu.async_copy(z_hbm.at[src_v.at[p]], zrs[p], sem_g)
      pltpu.async_copy(z_hbm.at[dst_v.at[p]], zcs[p], sem_g)

    lanes = lax.iota(jnp.int32, 16)
    colh = [((lanes + u) & 3) * 16 for u in range(4)]

    def step(c, b):
      zr_cur, zc_cur = zrs[b], zcs[b]
      nb = (b + nbuf - 1) % nbuf

      @pl.when(c + nbuf - 1 < nch)
      def _():
        pltpu.async_copy(z_hbm.at[src_v.at[c + nbuf - 1]], zrs[nb], sem_g)
        pltpu.async_copy(z_hbm.at[dst_v.at[c + nbuf - 1]], zcs[nb], sem_g)

      pltpu.make_async_copy(z_hbm.at[src_v.at[c]], zr_cur, sem_g).wait()
      pltpu.make_async_copy(z_hbm.at[dst_v.at[c]], zc_cur, sem_g).wait()
      def group(g, carry):
        rows = lanes + g * 16
        accs = [jnp.zeros((16,), jnp.float32) for _ in range(4)]
        for t in range(16):
          colt = (lanes + t) & 15
          for u in range(d // 16):
            col = colt + colh[u]
            accs[u] = accs[u] + (plsc.load_gather(zr_cur, [rows, col]) *
                                 plsc.load_gather(zc_cur, [rows, col]))
        out_v[c, pl.ds(g * 16, 16)] = (accs[0] + accs[1]) + (accs[2] + accs[3])
        return carry

      lax.fori_loop(0, ch // 16, group, 0)

    def body(k, carry):
      c = k * nbuf
      for off in range(nbuf):
        step(c + off, off)
      return carry

    lax.fori_loop(0, nch // nbuf, body, 0)
    for c in range((nch // nbuf) * nbuf, nch):
      step(c, c % nbuf)
    pltpu.sync_copy(out_v, out_hbm.at[wid])

  return decode_kernel


def _tc_y1(x, W1, deg0, deg1):
  blk = 1000
  grid = N // blk

  def body(xb, wb, d0b, d1b, yb):
    dinv = lax.rsqrt(d0b[...] + d1b[...] + 1.0)
    yb[...] = dinv * jnp.dot(xb[...], wb[...],
                             preferred_element_type=jnp.float32)

  return pl.pallas_call(
      body,
      grid=(grid,),
      in_specs=[
          pl.BlockSpec((blk, 128), lambda i: (i, 0)),
          pl.BlockSpec((128, 128), lambda i: (0, 0)),
          pl.BlockSpec((blk, 1), lambda i: (i, 0)),
          pl.BlockSpec((blk, 1), lambda i: (i, 0)),
      ],
      out_specs=pl.BlockSpec((blk, 128), lambda i: (i, 0)),
      out_shape=jax.ShapeDtypeStruct((N, 128), jnp.float32),
  )(x, W1, deg0, deg1)


def _tc_layer2_in(S1p, y1, deg0, deg1, b1, W2):
  blk = 1000
  grid = N // blk

  def body(sb, yb, d0b, d1b, bb, wb, ob):
    dinv = lax.rsqrt(d0b[...] + d1b[...] + 1.0)
    h = jax.nn.relu(dinv * (sb[0] + sb[1] + yb[...]) + bb[...])
    ob[...] = dinv * jnp.dot(h, wb[...], preferred_element_type=jnp.float32)

  return pl.pallas_call(
      body,
      grid=(grid,),
      in_specs=[
          pl.BlockSpec((2, blk, 128), lambda i: (0, i, 0)),
          pl.BlockSpec((blk, 128), lambda i: (i, 0)),
          pl.BlockSpec((blk, 1), lambda i: (i, 0)),
          pl.BlockSpec((blk, 1), lambda i: (i, 0)),
          pl.BlockSpec((1, 128), lambda i: (0, 0)),
          pl.BlockSpec((128, 64), lambda i: (0, 0)),
      ],
      out_specs=pl.BlockSpec((blk, 64), lambda i: (i, 0)),
      out_shape=jax.ShapeDtypeStruct((N, 64), jnp.float32),
  )(S1p, y1, deg0, deg1, b1, W2)


def _tc_z(S2p, y2, deg0, deg1, b2):
  blk = 1000
  grid = N // blk

  def body(sb, yb, d0b, d1b, bb, zb):
    dinv = lax.rsqrt(d0b[...] + d1b[...] + 1.0)
    zb[...] = dinv * (sb[0] + sb[1] + yb[...]) + bb[...]

  return pl.pallas_call(
      body,
      grid=(grid,),
      in_specs=[
          pl.BlockSpec((2, blk, 64), lambda i: (0, i, 0)),
          pl.BlockSpec((blk, 64), lambda i: (i, 0)),
          pl.BlockSpec((blk, 1), lambda i: (i, 0)),
          pl.BlockSpec((blk, 1), lambda i: (i, 0)),
          pl.BlockSpec((1, 64), lambda i: (0, 0)),
      ],
      out_specs=pl.BlockSpec((blk, 64), lambda i: (i, 0)),
      out_shape=jax.ShapeDtypeStruct((N, 64), jnp.float32),
  )(S2p, y2, deg0, deg1, b2)


_deg_kernel = _make_deg_kernel(80)
_edge_kernel_128 = _make_edge_kernel(128, 40)
_edge_kernel_64 = _make_edge_kernel(64, 80)
_decode_kernel = _make_decode_kernel(64, 80)


def kernel(x, edge_index, W1, b1, W2, b2):
  e = edge_index.shape[1]
  src80 = edge_index[0].reshape(NW, EPW // 80, 80)
  dst80 = edge_index[1].reshape(NW, EPW // 80, 80)
  src40 = edge_index[0].reshape(NW, EPW // 40, 40)
  dst40 = edge_index[1].reshape(NW, EPW // 40, 40)

  deg_part = _deg_kernel(dst80, jnp.zeros((RPS, 16), jnp.float32))
  deg0 = deg_part[0, :, 0:1]
  deg1 = deg_part[1, :, 0:1]

  y1 = _tc_y1(x, W1, deg0, deg1)
  S1p = _edge_kernel_128(y1, src40, dst40, jnp.zeros((RPS, 128), jnp.float32))
  y2 = _tc_layer2_in(S1p, y1, deg0, deg1, b1.reshape(1, 128), W2)
  S2p = _edge_kernel_64(y2, src80, dst80, jnp.zeros((RPS, 64), jnp.float32))
  z = _tc_z(S2p, y2, deg0, deg1, b2.reshape(1, 64))

  scores = _decode_kernel(z, src80, dst80)
  return scores.reshape(e)


# speedup vs baseline: 1.0749x; 1.0336x over previous
import functools

import jax
import jax.numpy as jnp
from jax import lax
from jax.experimental import pallas as pl
from jax.experimental.pallas import tpu as pltpu
from jax.experimental.pallas import tpu_sc as plsc

N = 10000
NC, NS, NW = 2, 16, 32
RPS = N // NS
EPW = 10000

_MESH = plsc.VectorSubcoreMesh(
    core_axis_name="c", subcore_axis_name="s", num_cores=NC, num_subcores=NS)

_SC_PARAMS = pltpu.CompilerParams(
    use_tc_tiling_on_sc=False, needs_layout_passes=False,
    disable_bounds_checks=True)


def _make_deg_kernel(ch):
  nch = EPW // ch

  @functools.partial(
      pl.kernel, mesh=_MESH,
      compiler_params=_SC_PARAMS,
      out_type=jax.ShapeDtypeStruct((NC, N, 16), jnp.float32),
      scratch_types=[
          pltpu.VMEM((nch, ch), jnp.int32),
          pltpu.VMEM((ch, 16), jnp.float32),
          pltpu.VMEM_SHARED((N, 16), jnp.float32),
      ],
  )
  def deg_kernel(dst_hbm, zeros_hbm, out_hbm, idx_v, ones_v, deg_sh):
    cid = lax.axis_index("c")
    sid = lax.axis_index("s")
    wid = cid * NS + sid
    pltpu.sync_copy(zeros_hbm, deg_sh.at[pl.ds(sid * RPS, RPS)])
    pltpu.sync_copy(dst_hbm.at[wid], idx_v)
    for r in range(ch):
      ones_v[r, :] = jnp.full((16,), 1.0, jnp.float32)
    plsc.subcore_barrier()

    def body(c, carry):
      pltpu.sync_copy(ones_v, deg_sh.at[idx_v.at[c]], add=True)
      return carry

    lax.fori_loop(0, nch, body, 0)
    plsc.subcore_barrier()
    pltpu.sync_copy(deg_sh.at[pl.ds(sid * RPS, RPS)],
                    out_hbm.at[cid, pl.ds(sid * RPS, RPS)])

  return deg_kernel


def _make_edge_kernel(d, ch):
  nch = EPW // ch
  nbuf = 5

  @functools.partial(
      pl.kernel, mesh=_MESH,
      compiler_params=_SC_PARAMS,
      out_type=jax.ShapeDtypeStruct((NC, N, d), jnp.float32),
      scratch_types=[
          pltpu.VMEM((nch, ch), jnp.int32),
          pltpu.VMEM((nch, ch), jnp.int32),
          [pltpu.VMEM((ch, d), jnp.float32) for _ in range(nbuf)],
          pltpu.VMEM_SHARED((N, d), jnp.float32),
          pltpu.SemaphoreType.DMA,
          pltpu.SemaphoreType.DMA,
      ],
  )
  def edge_kernel(y_hbm, src_hbm, dst_hbm, zeros_hbm, out_hbm,
                  src_v, dst_v, bufs, acc_sh, sem_g, sem_s):
    cid = lax.axis_index("c")
    sid = lax.axis_index("s")
    wid = cid * NS + sid
    pltpu.sync_copy(zeros_hbm, acc_sh.at[pl.ds(sid * RPS, RPS)])
    pltpu.sync_copy(src_hbm.at[wid], src_v)
    pltpu.sync_copy(dst_hbm.at[wid], dst_v)
    plsc.subcore_barrier()

    for p in range(nbuf - 2):
      pltpu.async_copy(y_hbm.at[src_v.at[p]], bufs[p], sem_g)

    def step(c, b):
      nb = (b + nbuf - 2) % nbuf

      @pl.when(c >= 2)
      def _():
        pltpu.make_async_copy(bufs[nb],
                              acc_sh.at[dst_v.at[c - 2]], sem_s).wait()

      @pl.when(c + nbuf - 2 < nch)
      def _():
        pltpu.async_copy(y_hbm.at[src_v.at[c + nbuf - 2]], bufs[nb], sem_g)

      pltpu.make_async_copy(y_hbm.at[src_v.at[c]], bufs[b], sem_g).wait()
      pltpu.async_copy(bufs[b], acc_sh.at[dst_v.at[c]], sem_s, add=True)

    def body(k, carry):
      c = k * nbuf
      for off in range(nbuf):
        step(c + off, off)
      return carry

    lax.fori_loop(0, nch // nbuf, body, 0)
    for c in range((nch // nbuf) * nbuf, nch):
      step(c, c % nbuf)
    pltpu.make_async_copy(bufs[(nch - 2) % nbuf],
                          acc_sh.at[dst_v.at[nch - 2]], sem_s).wait()
    pltpu.make_async_copy(bufs[(nch - 1) % nbuf],
                          acc_sh.at[dst_v.at[nch - 1]], sem_s).wait()
    plsc.subcore_barrier()
    pltpu.sync_copy(acc_sh.at[pl.ds(sid * RPS, RPS)],
                    out_hbm.at[cid, pl.ds(sid * RPS, RPS)])

  return edge_kernel


def _make_decode_kernel(d, ch):
  nch = EPW // ch

  nbuf = 3

  @functools.partial(
      pl.kernel, mesh=_MESH,
      compiler_params=_SC_PARAMS,
      out_type=jax.ShapeDtypeStruct((NW, nch, ch), jnp.float32),
      scratch_types=[
          pltpu.VMEM((nch, ch), jnp.int32),
          pltpu.VMEM((nch, ch), jnp.int32),
          [pltpu.VMEM((ch, d), jnp.float32) for _ in range(nbuf)],
          [pltpu.VMEM((ch, d), jnp.float32) for _ in range(nbuf)],
          pltpu.VMEM((nch, ch), jnp.float32),
          pltpu.SemaphoreType.DMA,
      ],
  )
  def decode_kernel(z_hbm, src_hbm, dst_hbm, out_hbm,
                    src_v, dst_v, zrs, zcs, out_v, sem_g):
    cid = lax.axis_index("c")
    sid = lax.axis_index("s")
    wid = cid * NS + sid
    pltpu.sync_copy(src_hbm.at[wid], src_v)
    pltpu.sync_copy(dst_hbm.at[wid], dst_v)

    for p in range(nbuf - 1):
      pltpu.async_copy(z_hbm.at[src_v.at[p]], zrs[p], sem_g)
      pltpu.async_copy(z_hbm.at[dst_v.at[p]], zcs[p], sem_g)

    lanes = lax.iota(jnp.int32, 16)
    colh = [((lanes + u) & 3) * 16 for u in range(4)]

    def step(c, b):
      zr_cur, zc_cur = zrs[b], zcs[b]
      nb = (b + nbuf - 1) % nbuf

      @pl.when(c + nbuf - 1 < nch)
      def _():
        pltpu.async_copy(z_hbm.at[src_v.at[c + nbuf - 1]], zrs[nb], sem_g)
        pltpu.async_copy(z_hbm.at[dst_v.at[c + nbuf - 1]], zcs[nb], sem_g)

      pltpu.make_async_copy(z_hbm.at[src_v.at[c]], zr_cur, sem_g).wait()
      pltpu.make_async_copy(z_hbm.at[dst_v.at[c]], zc_cur, sem_g).wait()
      def group(g, carry):
        rows = lanes + g * 16
        accs = [jnp.zeros((16,), jnp.float32) for _ in range(4)]
        for t in range(16):
          colt = (lanes + t) & 15
          for u in range(d // 16):
            col = colt + colh[u]
            accs[u] = accs[u] + (plsc.load_gather(zr_cur, [rows, col]) *
                                 plsc.load_gather(zc_cur, [rows, col]))
        out_v[c, pl.ds(g * 16, 16)] = (accs[0] + accs[1]) + (accs[2] + accs[3])
        return carry

      lax.fori_loop(0, ch // 16, group, 0)

    def body(k, carry):
      c = k * nbuf
      for off in range(nbuf):
        step(c + off, off)
      return carry

    lax.fori_loop(0, nch // nbuf, body, 0)
    for c in range((nch // nbuf) * nbuf, nch):
      step(c, c % nbuf)
    pltpu.sync_copy(out_v, out_hbm.at[wid])

  return decode_kernel


def _tc_y1(x, W1, deg0, deg1):
  blk = 1000
  grid = N // blk

  def body(xb, wb, d0b, d1b, yb):
    dinv = lax.rsqrt(d0b[...] + d1b[...] + 1.0)
    yb[...] = dinv * jnp.dot(xb[...], wb[...],
                             preferred_element_type=jnp.float32)

  return pl.pallas_call(
      body,
      grid=(grid,),
      in_specs=[
          pl.BlockSpec((blk, 128), lambda i: (i, 0)),
          pl.BlockSpec((128, 128), lambda i: (0, 0)),
          pl.BlockSpec((blk, 1), lambda i: (i, 0)),
          pl.BlockSpec((blk, 1), lambda i: (i, 0)),
      ],
      out_specs=pl.BlockSpec((blk, 128), lambda i: (i, 0)),
      out_shape=jax.ShapeDtypeStruct((N, 128), jnp.float32),
  )(x, W1, deg0, deg1)


def _tc_layer2_in(S1p, y1, deg0, deg1, b1, W2):
  blk = 1000
  grid = N // blk

  def body(sb, yb, d0b, d1b, bb, wb, ob):
    dinv = lax.rsqrt(d0b[...] + d1b[...] + 1.0)
    h = jax.nn.relu(dinv * (sb[0] + sb[1] + yb[...]) + bb[...])
    ob[...] = dinv * jnp.dot(h, wb[...], preferred_element_type=jnp.float32)

  return pl.pallas_call(
      body,
      grid=(grid,),
      in_specs=[
          pl.BlockSpec((2, blk, 128), lambda i: (0, i, 0)),
          pl.BlockSpec((blk, 128), lambda i: (i, 0)),
          pl.BlockSpec((blk, 1), lambda i: (i, 0)),
          pl.BlockSpec((blk, 1), lambda i: (i, 0)),
          pl.BlockSpec((1, 128), lambda i: (0, 0)),
          pl.BlockSpec((128, 64), lambda i: (0, 0)),
      ],
      out_specs=pl.BlockSpec((blk, 64), lambda i: (i, 0)),
      out_shape=jax.ShapeDtypeStruct((N, 64), jnp.float32),
  )(S1p, y1, deg0, deg1, b1, W2)


def _tc_z(S2p, y2, deg0, deg1, b2):
  blk = 1000
  grid = N // blk

  def body(sb, yb, d0b, d1b, bb, zb):
    dinv = lax.rsqrt(d0b[...] + d1b[...] + 1.0)
    zb[...] = dinv * (sb[0] + sb[1] + yb[...]) + bb[...]

  return pl.pallas_call(
      body,
      grid=(grid,),
      in_specs=[
          pl.BlockSpec((2, blk, 64), lambda i: (0, i, 0)),
          pl.BlockSpec((blk, 64), lambda i: (i, 0)),
          pl.BlockSpec((blk, 1), lambda i: (i, 0)),
          pl.BlockSpec((blk, 1), lambda i: (i, 0)),
          pl.BlockSpec((1, 64), lambda i: (0, 0)),
      ],
      out_specs=pl.BlockSpec((blk, 64), lambda i: (i, 0)),
      out_shape=jax.ShapeDtypeStruct((N, 64), jnp.float32),
  )(S2p, y2, deg0, deg1, b2)


_deg_kernel = _make_deg_kernel(80)
_edge_kernel_128 = _make_edge_kernel(128, 40)
_edge_kernel_64 = _make_edge_kernel(64, 80)
_decode_kernel = _make_decode_kernel(64, 80)


def kernel(x, edge_index, W1, b1, W2, b2):
  e = edge_index.shape[1]
  src80 = edge_index[0].reshape(NW, EPW // 80, 80)
  dst80 = edge_index[1].reshape(NW, EPW // 80, 80)
  src40 = edge_index[0].reshape(NW, EPW // 40, 40)
  dst40 = edge_index[1].reshape(NW, EPW // 40, 40)

  deg_part = _deg_kernel(dst80, jnp.zeros((RPS, 16), jnp.float32))
  deg0 = deg_part[0, :, 0:1]
  deg1 = deg_part[1, :, 0:1]

  y1 = _tc_y1(x, W1, deg0, deg1)
  S1p = _edge_kernel_128(y1, src40, dst40, jnp.zeros((RPS, 128), jnp.float32))
  y2 = _tc_layer2_in(S1p, y1, deg0, deg1, b1.reshape(1, 128), W2)
  S2p = _edge_kernel_64(y2, src80, dst80, jnp.zeros((RPS, 64), jnp.float32))
  z = _tc_z(S2p, y2, deg0, deg1, b2.reshape(1, 64))

  scores = _decode_kernel(z, src80, dst80)
  return scores.reshape(e)


# speedup vs baseline: 1.0954x; 1.0191x over previous
import functools

import jax
import jax.numpy as jnp
from jax import lax
from jax.experimental import pallas as pl
from jax.experimental.pallas import tpu as pltpu
from jax.experimental.pallas import tpu_sc as plsc

N = 10000
NC, NS, NW = 2, 16, 32
RPS = N // NS
EPW = 10000

_MESH = plsc.VectorSubcoreMesh(
    core_axis_name="c", subcore_axis_name="s", num_cores=NC, num_subcores=NS)

_SC_PARAMS = pltpu.CompilerParams(
    use_tc_tiling_on_sc=False, needs_layout_passes=False,
    disable_bounds_checks=True)


def _make_deg_kernel(ch):
  nch = EPW // ch

  @functools.partial(
      pl.kernel, mesh=_MESH,
      compiler_params=_SC_PARAMS,
      out_type=jax.ShapeDtypeStruct((NC, N, 16), jnp.float32),
      scratch_types=[
          pltpu.VMEM((nch, ch), jnp.int32),
          pltpu.VMEM((ch, 16), jnp.float32),
          pltpu.VMEM_SHARED((N, 16), jnp.float32),
          pltpu.SemaphoreType.DMA,
      ],
  )
  def deg_kernel(dst_hbm, zeros_hbm, out_hbm, idx_v, ones_v, deg_sh, sem_s):
    cid = lax.axis_index("c")
    sid = lax.axis_index("s")
    wid = cid * NS + sid
    pltpu.sync_copy(zeros_hbm, deg_sh.at[pl.ds(sid * RPS, RPS)])
    pltpu.sync_copy(dst_hbm.at[wid], idx_v)
    for r in range(ch):
      ones_v[r, :] = jnp.full((16,), 1.0, jnp.float32)
    plsc.subcore_barrier()

    def body(c, carry):
      @pl.when(c >= 4)
      def _():
        pltpu.make_async_copy(ones_v, deg_sh.at[idx_v.at[c - 4]], sem_s).wait()
      pltpu.async_copy(ones_v, deg_sh.at[idx_v.at[c]], sem_s, add=True)
      return carry

    lax.fori_loop(0, nch, body, 0)
    for c in range(nch - 4, nch):
      pltpu.make_async_copy(ones_v, deg_sh.at[idx_v.at[c]], sem_s).wait()
    plsc.subcore_barrier()
    pltpu.sync_copy(deg_sh.at[pl.ds(sid * RPS, RPS)],
                    out_hbm.at[cid, pl.ds(sid * RPS, RPS)])

  return deg_kernel


def _make_edge_kernel(d, ch):
  nch = EPW // ch
  nbuf = 5

  @functools.partial(
      pl.kernel, mesh=_MESH,
      compiler_params=_SC_PARAMS,
      out_type=jax.ShapeDtypeStruct((NC, N, d), jnp.float32),
      scratch_types=[
          pltpu.VMEM((nch, ch), jnp.int32),
          pltpu.VMEM((nch, ch), jnp.int32),
          [pltpu.VMEM((ch, d), jnp.float32) for _ in range(nbuf)],
          pltpu.VMEM_SHARED((N, d), jnp.float32),
          pltpu.SemaphoreType.DMA,
          pltpu.SemaphoreType.DMA,
      ],
  )
  def edge_kernel(y_hbm, src_hbm, dst_hbm, zeros_hbm, out_hbm,
                  src_v, dst_v, bufs, acc_sh, sem_g, sem_s):
    cid = lax.axis_index("c")
    sid = lax.axis_index("s")
    wid = cid * NS + sid
    pltpu.sync_copy(zeros_hbm, acc_sh.at[pl.ds(sid * RPS, RPS)])
    pltpu.sync_copy(src_hbm.at[wid], src_v)
    pltpu.sync_copy(dst_hbm.at[wid], dst_v)
    plsc.subcore_barrier()

    for p in range(nbuf - 2):
      pltpu.async_copy(y_hbm.at[src_v.at[p]], bufs[p], sem_g)

    def step(c, b):
      nb = (b + nbuf - 2) % nbuf

      @pl.when(c >= 2)
      def _():
        pltpu.make_async_copy(bufs[nb],
                              acc_sh.at[dst_v.at[c - 2]], sem_s).wait()

      @pl.when(c + nbuf - 2 < nch)
      def _():
        pltpu.async_copy(y_hbm.at[src_v.at[c + nbuf - 2]], bufs[nb], sem_g)

      pltpu.make_async_copy(y_hbm.at[src_v.at[c]], bufs[b], sem_g).wait()
      pltpu.async_copy(bufs[b], acc_sh.at[dst_v.at[c]], sem_s, add=True)

    def body(k, carry):
      c = k * nbuf
      for off in range(nbuf):
        step(c + off, off)
      return carry

    lax.fori_loop(0, nch // nbuf, body, 0)
    for c in range((nch // nbuf) * nbuf, nch):
      step(c, c % nbuf)
    pltpu.make_async_copy(bufs[(nch - 2) % nbuf],
                          acc_sh.at[dst_v.at[nch - 2]], sem_s).wait()
    pltpu.make_async_copy(bufs[(nch - 1) % nbuf],
                          acc_sh.at[dst_v.at[nch - 1]], sem_s).wait()
    plsc.subcore_barrier()
    pltpu.sync_copy(acc_sh.at[pl.ds(sid * RPS, RPS)],
                    out_hbm.at[cid, pl.ds(sid * RPS, RPS)])

  return edge_kernel


def _make_decode_kernel(d, ch):
  nch = EPW // ch

  nbuf = 3

  @functools.partial(
      pl.kernel, mesh=_MESH,
      compiler_params=_SC_PARAMS,
      out_type=jax.ShapeDtypeStruct((NW, nch, ch), jnp.float32),
      scratch_types=[
          pltpu.VMEM((nch, ch), jnp.int32),
          pltpu.VMEM((nch, ch), jnp.int32),
          [pltpu.VMEM((ch, d), jnp.float32) for _ in range(nbuf)],
          [pltpu.VMEM((ch, d), jnp.float32) for _ in range(nbuf)],
          pltpu.VMEM((nch, ch), jnp.float32),
          pltpu.SemaphoreType.DMA,
      ],
  )
  def decode_kernel(z_hbm, src_hbm, dst_hbm, out_hbm,
                    src_v, dst_v, zrs, zcs, out_v, sem_g):
    cid = lax.axis_index("c")
    sid = lax.axis_index("s")
    wid = cid * NS + sid
    pltpu.sync_copy(src_hbm.at[wid], src_v)
    pltpu.sync_copy(dst_hbm.at[wid], dst_v)

    for p in range(nbuf - 1):
      pltpu.async_copy(z_hbm.at[src_v.at[p]], zrs[p], sem_g)
      pltpu.async_copy(z_hbm.at[dst_v.at[p]], zcs[p], sem_g)

    lanes = lax.iota(jnp.int32, 16)
    colh = [((lanes + u) & 3) * 16 for u in range(4)]

    def step(c, b):
      zr_cur, zc_cur = zrs[b], zcs[b]
      nb = (b + nbuf - 1) % nbuf

      @pl.when(c + nbuf - 1 < nch)
      def _():
        pltpu.async_copy(z_hbm.at[src_v.at[c + nbuf - 1]], zrs[nb], sem_g)
        pltpu.async_copy(z_hbm.at[dst_v.at[c + nbuf - 1]], zcs[nb], sem_g)

      pltpu.make_async_copy(z_hbm.at[src_v.at[c]], zr_cur, sem_g).wait()
      pltpu.make_async_copy(z_hbm.at[dst_v.at[c]], zc_cur, sem_g).wait()
      def group(g, carry):
        rows = lanes + g * 16
        accs = [jnp.zeros((16,), jnp.float32) for _ in range(4)]
        for t in range(16):
          colt = (lanes + t) & 15
          for u in range(d // 16):
            col = colt + colh[u]
            accs[u] = accs[u] + (plsc.load_gather(zr_cur, [rows, col]) *
                                 plsc.load_gather(zc_cur, [rows, col]))
        out_v[c, pl.ds(g * 16, 16)] = (accs[0] + accs[1]) + (accs[2] + accs[3])
        return carry

      lax.fori_loop(0, ch // 16, group, 0)

    def body(k, carry):
      c = k * nbuf
      for off in range(nbuf):
        step(c + off, off)
      return carry

    lax.fori_loop(0, nch // nbuf, body, 0)
    for c in range((nch // nbuf) * nbuf, nch):
      step(c, c % nbuf)
    pltpu.sync_copy(out_v, out_hbm.at[wid])

  return decode_kernel


def _tc_y1(x, W1, deg0, deg1):
  blk = 1000
  grid = N // blk

  def body(xb, wb, d0b, d1b, yb):
    dinv = lax.rsqrt(d0b[...] + d1b[...] + 1.0)
    yb[...] = dinv * jnp.dot(xb[...], wb[...],
                             preferred_element_type=jnp.float32)

  return pl.pallas_call(
      body,
      grid=(grid,),
      in_specs=[
          pl.BlockSpec((blk, 128), lambda i: (i, 0)),
          pl.BlockSpec((128, 128), lambda i: (0, 0)),
          pl.BlockSpec((blk, 1), lambda i: (i, 0)),
          pl.BlockSpec((blk, 1), lambda i: (i, 0)),
      ],
      out_specs=pl.BlockSpec((blk, 128), lambda i: (i, 0)),
      out_shape=jax.ShapeDtypeStruct((N, 128), jnp.float32),
  )(x, W1, deg0, deg1)


def _tc_layer2_in(S1p, y1, deg0, deg1, b1, W2):
  blk = 1000
  grid = N // blk

  def body(sb, yb, d0b, d1b, bb, wb, ob):
    dinv = lax.rsqrt(d0b[...] + d1b[...] + 1.0)
    h = jax.nn.relu(dinv * (sb[0] + sb[1] + yb[...]) + bb[...])
    ob[...] = dinv * jnp.dot(h, wb[...], preferred_element_type=jnp.float32)

  return pl.pallas_call(
      body,
      grid=(grid,),
      in_specs=[
          pl.BlockSpec((2, blk, 128), lambda i: (0, i, 0)),
          pl.BlockSpec((blk, 128), lambda i: (i, 0)),
          pl.BlockSpec((blk, 1), lambda i: (i, 0)),
          pl.BlockSpec((blk, 1), lambda i: (i, 0)),
          pl.BlockSpec((1, 128), lambda i: (0, 0)),
          pl.BlockSpec((128, 64), lambda i: (0, 0)),
      ],
      out_specs=pl.BlockSpec((blk, 64), lambda i: (i, 0)),
      out_shape=jax.ShapeDtypeStruct((N, 64), jnp.float32),
  )(S1p, y1, deg0, deg1, b1, W2)


def _tc_z(S2p, y2, deg0, deg1, b2):
  blk = 1000
  grid = N // blk

  def body(sb, yb, d0b, d1b, bb, zb):
    dinv = lax.rsqrt(d0b[...] + d1b[...] + 1.0)
    zb[...] = dinv * (sb[0] + sb[1] + yb[...]) + bb[...]

  return pl.pallas_call(
      body,
      grid=(grid,),
      in_specs=[
          pl.BlockSpec((2, blk, 64), lambda i: (0, i, 0)),
          pl.BlockSpec((blk, 64), lambda i: (i, 0)),
          pl.BlockSpec((blk, 1), lambda i: (i, 0)),
          pl.BlockSpec((blk, 1), lambda i: (i, 0)),
          pl.BlockSpec((1, 64), lambda i: (0, 0)),
      ],
      out_specs=pl.BlockSpec((blk, 64), lambda i: (i, 0)),
      out_shape=jax.ShapeDtypeStruct((N, 64), jnp.float32),
  )(S2p, y2, deg0, deg1, b2)


_deg_kernel = _make_deg_kernel(80)
_edge_kernel_128 = _make_edge_kernel(128, 40)
_edge_kernel_64 = _make_edge_kernel(64, 80)
_decode_kernel = _make_decode_kernel(64, 80)


def kernel(x, edge_index, W1, b1, W2, b2):
  e = edge_index.shape[1]
  src80 = edge_index[0].reshape(NW, EPW // 80, 80)
  dst80 = edge_index[1].reshape(NW, EPW // 80, 80)
  src40 = edge_index[0].reshape(NW, EPW // 40, 40)
  dst40 = edge_index[1].reshape(NW, EPW // 40, 40)

  deg_part = _deg_kernel(dst80, jnp.zeros((RPS, 16), jnp.float32))
  deg0 = deg_part[0, :, 0:1]
  deg1 = deg_part[1, :, 0:1]

  y1 = _tc_y1(x, W1, deg0, deg1)
  S1p = _edge_kernel_128(y1, src40, dst40, jnp.zeros((RPS, 128), jnp.float32))
  y2 = _tc_layer2_in(S1p, y1, deg0, deg1, b1.reshape(1, 128), W2)
  S2p = _edge_kernel_64(y2, src80, dst80, jnp.zeros((RPS, 64), jnp.float32))
  z = _tc_z(S2p, y2, deg0, deg1, b2.reshape(1, 64))

  scores = _decode_kernel(z, src80, dst80)
  return scores.reshape(e)
